# baseline (speedup 1.0000x reference)
import jax
import jax.numpy as jnp
import numpy as np
from jax import lax
from jax.experimental import pallas as pl
from jax.experimental.pallas import tpu as pltpu, tpu_sc as plsc

R, N, TOPK = 64, 32768, 256
L = 16
NC, NS = 2, 16
NW = NC * NS
ROWS_PER_W = R // NW
CH = N // L
HB = 256
UNROLL = 8

_SIGN = np.uint32(0x80000000)
_ONES = np.uint32(0xFFFFFFFF)


def _keys_u(u):
    return u ^ jnp.where(u >= _SIGN, _ONES, _SIGN)


def _inv_key(k):
    u = k ^ jnp.where(k >= _SIGN, _SIGN, _ONES)
    return lax.bitcast_convert_type(u, jnp.float32)


def _process_row(row_v, cand_v, hist_v):
    iota = lax.iota(jnp.int32, L)
    iota16 = iota * L
    ones = jnp.ones((L,), jnp.int32)
    zeros = jnp.zeros((L,), jnp.int32)

    def zero_hist():
        def zh(i, _):
            hist_v[pl.ds(i * L, L)] = zeros
            return 0
        lax.fori_loop(0, HB // L, zh, 0, unroll=4)

    def scan_buckets(kk):
        tot = hist_v[pl.ds(0, L)]
        for i in range(1, L):
            tot = tot + hist_v[pl.ds(i * L, L)]
        cum_ge = lax.rev(plsc.cumsum(lax.rev(tot, (0,))), (0,))
        sel = cum_ge >= kk
        dstar = jnp.max(jnp.where(sel, iota, -1))
        knext = jnp.max(jnp.where(sel, kk - (cum_ge - tot), np.int32(-(2**31))))
        return dstar, knext

    zero_hist()

    @plsc.parallel_loop(0, CH, unroll=UNROLL)
    def _(j):
        u = lax.bitcast_convert_type(row_v[pl.ds(j * L, L)], jnp.uint32)
        a = lax.shift_right_logical(u, jnp.broadcast_to(np.uint32(28), (L,)))
        flip = jnp.where(a >= 8, np.uint32(15), np.uint32(8))
        idx = iota16 + lax.bitcast_convert_type(a ^ flip, jnp.int32)
        plsc.addupdate_scatter(hist_v, [idx], ones)

    dstar, kk = scan_buckets(np.int32(TOPK))
    dstar_u = lax.convert_element_type(dstar, jnp.uint32)
    prefix = lax.shift_left(dstar_u, np.uint32(28))

    f_lo = _inv_key(prefix)
    f_hi = _inv_key(prefix + np.uint32(1 << 28))
    top_bucket = dstar == 15
    bot_bucket = dstar == 0

    @plsc.parallel_loop(0, CH, unroll=UNROLL, carry=dstar * 0)
    def n_cand(j, off):
        xv = row_v[pl.ds(j * L, L)]
        m = ((xv >= f_lo) | bot_bucket) & ((xv < f_hi) | top_bucket)
        plsc.store_compressed(
            cand_v.at[pl.ds(off, L)],
            lax.bitcast_convert_type(xv, jnp.uint32), mask=m)
        return off + plsc.all_reduce_population_count(m)[0]

    zero_hist()

    def h1body(j, _):
        kv = _keys_u(cand_v[pl.ds(j * L, L)])
        nd = jnp.bitwise_and(
            jnp.right_shift(kv, np.uint32(24)), np.uint32(15))
        idx = iota16 + lax.bitcast_convert_type(nd, jnp.int32)
        m = (j * L + iota) < n_cand
        plsc.addupdate_scatter(hist_v, [idx], ones, mask=m)
        return 0

    lax.fori_loop(0, (n_cand + (L - 1)) // L, h1body, 0)

    for l in range(1, 8):
        dstar, knext = scan_buckets(kk)
        dstar_u = lax.convert_element_type(dstar, jnp.uint32)
        prefix = prefix | lax.shift_left(dstar_u, np.uint32(28 - 4 * l))

        if l == 7:
            break

        zero_hist()
        sh_this = np.uint32(28 - 4 * l)
        sh_next = np.uint32(28 - 4 * (l + 1))

        def cbody(j, off, sh_this=sh_this, sh_next=sh_next,
                  dstar_u=dstar_u, lvl=l, n_src=n_cand):
            raw = cand_v[pl.ds(j * L, L)]
            kv = _keys_u(raw) if lvl == 1 else raw
            digit = jnp.bitwise_and(
                jnp.right_shift(kv, sh_this), np.uint32(15))
            m = (digit == dstar_u) & ((j * L + iota) < n_src)
            plsc.store_compressed(cand_v.at[pl.ds(off, L)], kv, mask=m)
            nd = jnp.bitwise_and(
                jnp.right_shift(kv, sh_next), np.uint32(15))
            idx = iota16 + lax.bitcast_convert_type(nd, jnp.int32)
            plsc.addupdate_scatter(hist_v, [idx], ones, mask=m)
            return off + plsc.all_reduce_population_count(m)[0]

        trip = (n_cand + (L - 1)) // L
        n_cand = lax.fori_loop(0, trip, cbody, np.int32(0))
        kk = knext

    thresh_f = _inv_key(prefix)

    @plsc.parallel_loop(0, CH, unroll=UNROLL)
    def _(j):
        sl = pl.ds(j * L, L)
        xv = row_v[sl]
        row_v[sl] = jnp.where(xv >= thresh_f, xv, 0.0)


def _body(x_hbm, out_hbm, row_a, row_b, cand_v, hist_v, sem_a, sem_b):
    wid = lax.axis_index("s") * NC + lax.axis_index("c")
    row0 = wid * ROWS_PER_W
    row1 = row0 + 1

    in0 = pltpu.async_copy(x_hbm.at[row0], row_a, sem_a)
    in1 = pltpu.async_copy(x_hbm.at[row1], row_b, sem_b)
    in0.wait()
    _process_row(row_a, cand_v, hist_v)
    out0 = pltpu.async_copy(row_a, out_hbm.at[row0], sem_a)
    in1.wait()
    _process_row(row_b, cand_v, hist_v)
    out0.wait()
    pltpu.async_copy(row_b, out_hbm.at[row1], sem_b).wait()


@jax.jit
def kernel(x):
    mesh = plsc.VectorSubcoreMesh(
        core_axis_name="c", subcore_axis_name="s",
        num_cores=NC, num_subcores=NS)
    return pl.kernel(
        _body,
        out_type=jax.ShapeDtypeStruct((R, N), jnp.float32),
        mesh=mesh,
        compiler_params=pltpu.CompilerParams(needs_layout_passes=False),
        scratch_types=[
            pltpu.VMEM((N,), jnp.float32),
            pltpu.VMEM((N,), jnp.float32),
            pltpu.VMEM((N,), jnp.uint32),
            pltpu.VMEM((HB,), jnp.int32),
            pltpu.SemaphoreType.DMA,
            pltpu.SemaphoreType.DMA,
        ],
    )(x)


# speedup vs baseline: 1.5516x; 1.5516x over previous
import jax
import jax.numpy as jnp
import numpy as np
from jax import lax
from jax.experimental import pallas as pl
from jax.experimental.pallas import tpu as pltpu, tpu_sc as plsc

R, N, TOPK = 64, 32768, 256
L = 16
NC, NS = 2, 16
NW = NC * NS
ROWS_PER_W = R // NW
CH = N // L
HB = 256
UNROLL = 8

_SIGN = np.uint32(0x80000000)
_ONES = np.uint32(0xFFFFFFFF)


def _keys_u(u):
    return u ^ jnp.where(u >= _SIGN, _ONES, _SIGN)


def _inv_key(k):
    u = k ^ jnp.where(k >= _SIGN, _SIGN, _ONES)
    return lax.bitcast_convert_type(u, jnp.float32)


def _process_row(row_v, cand_v, hist_v):
    iota = lax.iota(jnp.int32, L)
    iota16 = iota * L
    ones = jnp.ones((L,), jnp.int32)
    zeros = jnp.zeros((L,), jnp.int32)

    def zero_hist():
        def zh(i, _):
            hist_v[pl.ds(i * L, L)] = zeros
            return 0
        lax.fori_loop(0, HB // L, zh, 0, unroll=4)

    def scan_buckets(kk):
        tot = hist_v[pl.ds(0, L)]
        for i in range(1, L):
            tot = tot + hist_v[pl.ds(i * L, L)]
        cum_ge = lax.rev(plsc.cumsum(lax.rev(tot, (0,))), (0,))
        sel = cum_ge >= kk
        dstar = jnp.max(jnp.where(sel, iota, -1))
        knext = jnp.max(jnp.where(sel, kk - (cum_ge - tot), np.int32(-(2**31))))
        return dstar, knext

    zero_hist()

    @plsc.parallel_loop(0, CH, unroll=UNROLL)
    def _(j):
        u = lax.bitcast_convert_type(row_v[pl.ds(j * L, L)], jnp.uint32)
        a = lax.shift_right_logical(u, jnp.broadcast_to(np.uint32(28), (L,)))
        flip = jnp.where(a >= 8, np.uint32(15), np.uint32(8))
        idx = iota16 + lax.bitcast_convert_type(a ^ flip, jnp.int32)
        plsc.addupdate_scatter(hist_v, [idx], ones)

    dstar, kk = scan_buckets(np.int32(TOPK))
    dstar_u = lax.convert_element_type(dstar, jnp.uint32)
    prefix = lax.shift_left(dstar_u, np.uint32(28))

    zero_hist()

    @plsc.parallel_loop(0, CH, unroll=UNROLL, carry=dstar * 0)
    def n_cand(j, off):
        u = lax.bitcast_convert_type(row_v[pl.ds(j * L, L)], jnp.uint32)
        kv = _keys_u(u)
        digit = lax.shift_right_logical(
            kv, jnp.broadcast_to(np.uint32(28), (L,)))
        m = digit == dstar_u
        plsc.store_compressed(cand_v.at[pl.ds(off, L)], kv, mask=m)
        nd = jnp.bitwise_and(
            jnp.right_shift(kv, np.uint32(24)), np.uint32(15))
        idx = iota16 + lax.bitcast_convert_type(nd, jnp.int32)
        plsc.addupdate_scatter(hist_v, [idx], ones, mask=m)
        return off + plsc.all_reduce_population_count(m)[0]

    for l in range(1, 8):
        dstar, knext = scan_buckets(kk)
        dstar_u = lax.convert_element_type(dstar, jnp.uint32)
        prefix = prefix | lax.shift_left(dstar_u, np.uint32(28 - 4 * l))

        if l == 7:
            break

        zero_hist()
        sh_this = np.uint32(28 - 4 * l)
        sh_next = np.uint32(28 - 4 * (l + 1))

        def cbody(j, off, sh_this=sh_this, sh_next=sh_next,
                  dstar_u=dstar_u, lvl=l, n_src=n_cand):
            kv = cand_v[pl.ds(j * L, L)]
            digit = jnp.bitwise_and(
                jnp.right_shift(kv, sh_this), np.uint32(15))
            m = (digit == dstar_u) & ((j * L + iota) < n_src)
            plsc.store_compressed(cand_v.at[pl.ds(off, L)], kv, mask=m)
            nd = jnp.bitwise_and(
                jnp.right_shift(kv, sh_next), np.uint32(15))
            idx = iota16 + lax.bitcast_convert_type(nd, jnp.int32)
            plsc.addupdate_scatter(hist_v, [idx], ones, mask=m)
            return off + plsc.all_reduce_population_count(m)[0]

        trip = (n_cand + (L - 1)) // L
        n_cand = lax.fori_loop(0, trip, cbody, np.int32(0))
        kk = knext

    thresh_f = _inv_key(prefix)

    @plsc.parallel_loop(0, CH, unroll=UNROLL)
    def _(j):
        sl = pl.ds(j * L, L)
        xv = row_v[sl]
        row_v[sl] = jnp.where(xv >= thresh_f, xv, 0.0)


def _body(x_hbm, out_hbm, row_a, row_b, cand_v, hist_v, sem_a, sem_b):
    wid = lax.axis_index("s") * NC + lax.axis_index("c")
    row0 = wid * ROWS_PER_W
    row1 = row0 + 1

    in0 = pltpu.async_copy(x_hbm.at[row0], row_a, sem_a)
    in1 = pltpu.async_copy(x_hbm.at[row1], row_b, sem_b)
    in0.wait()
    _process_row(row_a, cand_v, hist_v)
    out0 = pltpu.async_copy(row_a, out_hbm.at[row0], sem_a)
    in1.wait()
    _process_row(row_b, cand_v, hist_v)
    out0.wait()
    pltpu.async_copy(row_b, out_hbm.at[row1], sem_b).wait()


@jax.jit
def kernel(x):
    mesh = plsc.VectorSubcoreMesh(
        core_axis_name="c", subcore_axis_name="s",
        num_cores=NC, num_subcores=NS)
    return pl.kernel(
        _body,
        out_type=jax.ShapeDtypeStruct((R, N), jnp.float32),
        mesh=mesh,
        compiler_params=pltpu.CompilerParams(needs_layout_passes=False),
        scratch_types=[
            pltpu.VMEM((N,), jnp.float32),
            pltpu.VMEM((N,), jnp.float32),
            pltpu.VMEM((N,), jnp.uint32),
            pltpu.VMEM((HB,), jnp.int32),
            pltpu.SemaphoreType.DMA,
            pltpu.SemaphoreType.DMA,
        ],
    )(x)
